# Initial kernel scaffold; baseline (speedup 1.0000x reference)
import jax
import jax.numpy as jnp
from jax.experimental import pallas as pl


def kernel(node_embedding, nbr_idx, nbr_mask):
    raise NotImplementedError("write your pallas kernel here")


# speedup vs baseline: 27.0858x; 27.0858x over previous
import jax
import jax.numpy as jnp
from jax.experimental import pallas as pl


def _get_node_k_kernel(emb_ref, idx_ref, mask_ref, out_ref):
    _, At, D = emb_ref.shape
    _, A, Nbr = idx_ref.shape

    emb = emb_ref[0]
    idx = idx_ref[0]
    msk = mask_ref[0]

    iota_at = jax.lax.broadcasted_iota(jnp.int32, (A, Nbr, At), 2)
    hot = (iota_at == idx[:, :, None]) & (msk[:, :, None] != 0)
    oh = jnp.where(hot, 1.0, 0.0).reshape(A * Nbr, At)
    g = jnp.dot(oh, emb, preferred_element_type=jnp.float32)
    g = g.reshape(A, Nbr, D)

    for i in range(Nbr):
        if i > 0:
            out_ref[0, :, i, :i, :] = g[:, :i, :]
        if i < Nbr - 1:
            out_ref[0, :, i, i:, :] = g[:, i + 1 :, :]


def kernel(node_embedding, nbr_idx, nbr_mask):
    B, At, D = node_embedding.shape
    Nbr = nbr_idx.shape[-1]
    A_BLK = 32

    grid = (B, At // A_BLK)
    out = pl.pallas_call(
        _get_node_k_kernel,
        grid=grid,
        in_specs=[
            pl.BlockSpec((1, At, D), lambda b, a: (b, 0, 0)),
            pl.BlockSpec((1, A_BLK, Nbr), lambda b, a: (b, a, 0)),
            pl.BlockSpec((1, A_BLK, Nbr), lambda b, a: (b, a, 0)),
        ],
        out_specs=pl.BlockSpec(
            (1, A_BLK, Nbr, Nbr - 1, D), lambda b, a: (b, a, 0, 0, 0)
        ),
        out_shape=jax.ShapeDtypeStruct((B, At, Nbr, Nbr - 1, D), jnp.float32),
    )(node_embedding, nbr_idx, nbr_mask)
    return out


# speedup vs baseline: 27.4154x; 1.0122x over previous
import jax
import jax.numpy as jnp
from jax.experimental import pallas as pl


def _get_node_k_kernel(emb_ref, idx_ref, mask_ref, out_ref):
    _, At, D = emb_ref.shape
    _, A, Nbr = idx_ref.shape

    emb = emb_ref[0]
    idx = idx_ref[0]
    msk = mask_ref[0]

    iota_at = jax.lax.broadcasted_iota(jnp.int32, (A, Nbr, At), 2)
    hot = (iota_at == idx[:, :, None]) & (msk[:, :, None] != 0)
    oh = jnp.where(hot, 1.0, 0.0).reshape(A * Nbr, At)
    g = jnp.dot(oh, emb, preferred_element_type=jnp.float32)
    g = g.reshape(A, Nbr, D)

    for i in range(Nbr):
        if i > 0:
            out_ref[0, :, i, :i, :] = g[:, :i, :]
        if i < Nbr - 1:
            out_ref[0, :, i, i:, :] = g[:, i + 1 :, :]


def kernel(node_embedding, nbr_idx, nbr_mask):
    B, At, D = node_embedding.shape
    Nbr = nbr_idx.shape[-1]
    A_BLK = 64

    grid = (B, At // A_BLK)
    out = pl.pallas_call(
        _get_node_k_kernel,
        grid=grid,
        in_specs=[
            pl.BlockSpec((1, At, D), lambda b, a: (b, 0, 0)),
            pl.BlockSpec((1, A_BLK, Nbr), lambda b, a: (b, a, 0)),
            pl.BlockSpec((1, A_BLK, Nbr), lambda b, a: (b, a, 0)),
        ],
        out_specs=pl.BlockSpec(
            (1, A_BLK, Nbr, Nbr - 1, D), lambda b, a: (b, a, 0, 0, 0)
        ),
        out_shape=jax.ShapeDtypeStruct((B, At, Nbr, Nbr - 1, D), jnp.float32),
    )(node_embedding, nbr_idx, nbr_mask)
    return out
